# baseline (speedup 1.0000x reference)
import functools

import jax
import jax.numpy as jnp
from jax import lax
from jax.experimental import pallas as pl
from jax.experimental.pallas import tpu as pltpu
from jax.experimental.pallas import tpu_sc as plsc

_CHUNK = 128
_L = 16


@functools.cache
def _build_gather(B: int, V: int, D: int):
    info = plsc.get_sparse_core_info()
    NC, NS = info.num_cores, info.num_subcores
    NW = NC * NS
    assert B % (NW * _L) == 0
    assert D == 4
    b_per_w = B // NW
    e_per_w = b_per_w * D
    assert e_per_w % _CHUNK == 0
    r_stride = (V // NS) & ~7
    r_per_t = V - (NS - 1) * r_stride
    assert r_per_t >= r_stride and (r_per_t * D) % 8 == 0
    w_per_t = r_per_t * D
    mesh = plsc.VectorSubcoreMesh(core_axis_name="c", subcore_axis_name="s")

    @functools.partial(
        pl.kernel,
        mesh=mesh,
        out_type=jax.ShapeDtypeStruct((B, D), jnp.float32),
        compiler_params=pltpu.CompilerParams(
            use_tc_tiling_on_sc=False, needs_layout_passes=False
        ),
        scratch_types=[
            pltpu.VMEM((r_per_t, D), jnp.float32),
            pltpu.VMEM((w_per_t,), jnp.float32),
            pltpu.VMEM_SHARED((V * D,), jnp.float32),
            pltpu.VMEM((b_per_w,), jnp.int32),
            pltpu.VMEM((e_per_w,), jnp.int32),
            pltpu.VMEM((e_per_w,), jnp.float32),
            pltpu.VMEM((b_per_w, D), jnp.float32),
            pltpu.SemaphoreType.DMA,
        ],
    )
    def gather_k(ids_hbm, table_hbm, out_hbm, buf4, bufflat, shared, idx_v,
                 eidx_v, vals_v, vals2_v, sem):
        cid = lax.axis_index("c")
        sid = lax.axis_index("s")
        wid = sid * NC + cid
        lanes = lax.iota(jnp.int32, _L)
        base = wid * b_per_w

        r0 = sid * r_stride
        detile = pltpu.async_copy(table_hbm.at[pl.ds(r0, r_per_t)], buf4, sem)

        pltpu.sync_copy(ids_hbm.at[pl.ds(base, b_per_w)], idx_v)
        for k in range(b_per_w // _L):
            v4 = idx_v[pl.ds(k * _L, _L)] * D
            pos = lanes * D + (k * _L * D)
            for c in range(D):
                plsc.store_scatter(eidx_v, [pos + c], v4 + c)

        detile.wait()
        n_rep = (w_per_t + _L - 1) // _L
        cap = jnp.int32(w_per_t - 1)

        def repack_body(k, carry):
            e = jnp.minimum(lanes + k * _L, cap)
            v = plsc.load_gather(buf4, [e >> 2, e & 3])
            plsc.store_scatter(bufflat, [e], v)
            return carry

        lax.fori_loop(0, n_rep, repack_body, 0, unroll=4)
        pltpu.sync_copy(bufflat, shared.at[pl.ds(r0 * D, w_per_t)])
        plsc.subcore_barrier()

        copies = [
            pltpu.async_copy(
                shared.at[eidx_v.at[pl.ds(j * _CHUNK, _CHUNK)]],
                vals_v.at[pl.ds(j * _CHUNK, _CHUNK)],
                sem,
            )
            for j in range(e_per_w // _CHUNK)
        ]
        for cpy in copies:
            cpy.wait()

        def out_repack(k, carry):
            e = lanes + k * _L
            v = vals_v[pl.ds(k * _L, _L)]
            plsc.store_scatter(vals2_v, [e >> 2, e & 3], v)
            return carry

        lax.fori_loop(0, e_per_w // _L, out_repack, 0, unroll=4)
        pltpu.sync_copy(vals2_v, out_hbm.at[pl.ds(base, b_per_w)])

    return gather_k


def kernel(ids, data):
    B, = ids.shape
    V, D = data.shape
    gather_k = _build_gather(B, V, D)
    return gather_k(ids.astype(jnp.int32), data)


# speedup vs baseline: 1.2517x; 1.2517x over previous
import functools

import jax
import jax.numpy as jnp
from jax import lax
from jax.experimental import pallas as pl
from jax.experimental.pallas import tpu as pltpu
from jax.experimental.pallas import tpu_sc as plsc

_CHUNK = 128
_L = 16
_W = 128


@functools.cache
def _build_gather(B: int, V: int, D: int):
    info = plsc.get_sparse_core_info()
    NC, NS = info.num_cores, info.num_subcores
    NW = NC * NS
    rpp = _W // D
    assert B % (NW * _CHUNK) == 0 and V % rpp == 0
    assert D == 4
    b_per_w = B // NW
    e_per_w = b_per_w * D
    n_chunk = b_per_w // _CHUNK
    o_rows = e_per_w // _W
    mesh = plsc.VectorSubcoreMesh(core_axis_name="c", subcore_axis_name="s")

    @functools.partial(
        pl.kernel,
        mesh=mesh,
        out_type=jax.ShapeDtypeStruct((B * D // _W, _W), jnp.float32),
        compiler_params=pltpu.CompilerParams(
            use_tc_tiling_on_sc=True, needs_layout_passes=False
        ),
        scratch_types=[
            pltpu.VMEM((b_per_w,), jnp.int32),
            pltpu.VMEM((b_per_w,), jnp.int32),
            pltpu.VMEM((b_per_w, _W), jnp.float32),
            pltpu.VMEM((o_rows, _W), jnp.float32),
            pltpu.SemaphoreType.DMA,
        ],
    )
    def gather_k(ids_hbm, packed_hbm, out_hbm, idx_v, prow_v, rows_v,
                 vals_v, sem):
        wid = lax.axis_index("s") * NC + lax.axis_index("c")
        base = wid * b_per_w
        lanes = lax.iota(jnp.int32, _L)

        pltpu.sync_copy(ids_hbm.at[pl.ds(base, b_per_w)], idx_v)
        for k in range(b_per_w // _L):
            prow_v[pl.ds(k * _L, _L)] = idx_v[pl.ds(k * _L, _L)] >> 5

        copies = [
            pltpu.async_copy(
                packed_hbm.at[prow_v.at[pl.ds(j * _CHUNK, _CHUNK)]],
                rows_v.at[pl.ds(j * _CHUNK, _CHUNK)],
                sem,
            )
            for j in range(n_chunk)
        ]
        for cpy in copies:
            cpy.wait()

        def compact(k, carry):
            e = lanes + k * _L
            i = e >> 2
            idsv = plsc.load_gather(idx_v, [i])
            colv = ((idsv & (rpp - 1)) << 2) + (e & 3)
            v = plsc.load_gather(rows_v, [i, colv])
            plsc.store_scatter(vals_v, [e >> 7, e & (_W - 1)], v)
            return carry

        lax.fori_loop(0, e_per_w // _L, compact, 0, unroll=4)
        pltpu.sync_copy(vals_v, out_hbm.at[pl.ds(wid * o_rows, o_rows)])

    return gather_k


def kernel(ids, data):
    B, = ids.shape
    V, D = data.shape
    gather_k = _build_gather(B, V, D)
    packed = data.reshape(V * D // _W, _W)
    out = gather_k(ids.astype(jnp.int32), packed)
    return out.reshape(B, D)
